# baseline (speedup 1.0000x reference)
import functools

import jax
import jax.numpy as jnp
from jax import lax
from jax.experimental import pallas as pl
from jax.experimental.pallas import tpu as pltpu
from jax.experimental.pallas import tpu_sc as plsc

EMB_D = 128
BATCH = 16384
NUM_CORES = 2
NUM_SUBCORES = 16
NUM_WORKERS = NUM_CORES * NUM_SUBCORES
B_PER_W = BATCH // NUM_WORKERS
LANES = 16
CHUNKS = EMB_D // LANES
GROUPS = B_PER_W // LANES
NB = 2
GROUPS_PER_NB = GROUPS // NB
WORDS_PER_NB = B_PER_W * EMB_D // NB


def _build():
    mesh = plsc.VectorSubcoreMesh(core_axis_name="c", subcore_axis_name="s")

    @functools.partial(
        pl.kernel,
        mesh=mesh,
        out_type=jax.ShapeDtypeStruct((BATCH * EMB_D,), jnp.float32),
        scratch_types=[
            pltpu.VMEM((B_PER_W,), jnp.int32),
            pltpu.VMEM((3 * EMB_D,), jnp.float32),
            pltpu.VMEM((B_PER_W * EMB_D,), jnp.float32),
            pltpu.SemaphoreType.DMA,
            pltpu.SemaphoreType.DMA,
        ],
    )
    def lookup_kernel(labels_hbm, table_hbm, out_hbm, idx_v, tab_v, rows_v,
                      sem_in, sem_out):
        wid = lax.axis_index("s") * NUM_CORES + lax.axis_index("c")
        base = wid * B_PER_W
        cp_lab = pltpu.async_copy(
            labels_hbm.at[pl.ds(base, B_PER_W)], idx_v, sem_in)
        cp_tab = pltpu.async_copy(table_hbm, tab_v, sem_in)
        cp_lab.wait()
        cp_tab.wait()
        rows = [
            [tab_v[pl.ds(v * EMB_D + LANES * c, LANES)] for c in range(CHUNKS)]
            for v in range(3)
        ]
        d1 = [rows[1][c] - rows[0][c] for c in range(CHUNKS)]
        d2 = [rows[2][c] - rows[0][c] for c in range(CHUNKS)]

        def body(g, carry):
            lbl16 = idx_v[pl.ds(g * LANES, LANES)]
            for j in range(LANES):
                lbl = lbl16[j]
                e = g * LANES + j
                for v in range(3):
                    @pl.when(lbl == v)
                    def _(v=v, e=e):
                        for c in range(CHUNKS):
                            rows_v[pl.ds(e * EMB_D + LANES * c, LANES)] = rows[v][c]
            return carry

        out_cps = []
        for nb in range(NB):
            lax.fori_loop(nb * GROUPS_PER_NB, (nb + 1) * GROUPS_PER_NB,
                          body, 0)
            out_cps.append(pltpu.async_copy(
                rows_v.at[pl.ds(nb * WORDS_PER_NB, WORDS_PER_NB)],
                out_hbm.at[pl.ds(base * EMB_D + nb * WORDS_PER_NB, WORDS_PER_NB)],
                sem_out))
        for cp in out_cps:
            cp.wait()

    return lookup_kernel


_lookup = _build()


def kernel(labels, table):
    out = _lookup(labels, table.reshape(-1))
    return out.reshape(BATCH, 1, EMB_D)


# speedup vs baseline: 1.4139x; 1.4139x over previous
import functools

import jax
import jax.numpy as jnp
from jax import lax
from jax.experimental import pallas as pl
from jax.experimental.pallas import tpu as pltpu
from jax.experimental.pallas import tpu_sc as plsc

EMB_D = 128
BATCH = 16384
NUM_CORES = 2
NUM_SUBCORES = 16
NUM_WORKERS = NUM_CORES * NUM_SUBCORES
B_PER_W = BATCH // NUM_WORKERS
LANES = 16


def _build():
    mesh = plsc.VectorSubcoreMesh(core_axis_name="c", subcore_axis_name="s")

    @functools.partial(
        pl.kernel,
        mesh=mesh,
        out_type=jax.ShapeDtypeStruct((BATCH, EMB_D), jnp.float32),
        scratch_types=[
            pltpu.VMEM((B_PER_W,), jnp.int32),
            pltpu.VMEM((B_PER_W, EMB_D), jnp.float32),
            pltpu.VMEM_SHARED((3, EMB_D), jnp.float32),
            pltpu.SemaphoreType.DMA,
            pltpu.SemaphoreType.DMA,
        ],
    )
    def lookup_kernel(labels_hbm, table_hbm, out_hbm, idx_v, rows_v, tab_sh,
                      sem_in, sem_out):
        sid = lax.axis_index("s")
        wid = sid * NUM_CORES + lax.axis_index("c")
        base = wid * B_PER_W
        cp_lab = pltpu.async_copy(
            labels_hbm.at[pl.ds(base, B_PER_W)], idx_v, sem_in)

        @pl.when(sid == 0)
        def _():
            pltpu.sync_copy(table_hbm, tab_sh)

        cp_lab.wait()
        plsc.subcore_barrier()
        pltpu.async_copy(tab_sh.at[idx_v], rows_v, sem_out).wait()
        pltpu.sync_copy(rows_v, out_hbm.at[pl.ds(base, B_PER_W)])

    return lookup_kernel


_lookup = _build()


def kernel(labels, table):
    out = _lookup(labels, table)
    return out.reshape(BATCH, 1, EMB_D)


# speedup vs baseline: 1.4507x; 1.0260x over previous
import functools

import jax
import jax.numpy as jnp
from jax import lax
from jax.experimental import pallas as pl
from jax.experimental.pallas import tpu as pltpu
from jax.experimental.pallas import tpu_sc as plsc

EMB_D = 128
BATCH = 16384
NUM_CORES = 2
NUM_SUBCORES = 16
NUM_WORKERS = NUM_CORES * NUM_SUBCORES
B_PER_W = BATCH // NUM_WORKERS
HALF = B_PER_W // 2


def _build():
    mesh = plsc.VectorSubcoreMesh(core_axis_name="c", subcore_axis_name="s")

    @functools.partial(
        pl.kernel,
        mesh=mesh,
        out_type=jax.ShapeDtypeStruct((BATCH, EMB_D), jnp.float32),
        scratch_types=[
            pltpu.VMEM((HALF,), jnp.int32),
            pltpu.VMEM((HALF,), jnp.int32),
            pltpu.VMEM((2, HALF, EMB_D), jnp.float32),
            pltpu.VMEM_SHARED((3, EMB_D), jnp.float32),
            pltpu.SemaphoreType.DMA,
            pltpu.SemaphoreType.DMA,
            pltpu.SemaphoreType.DMA,
        ],
    )
    def lookup_kernel(labels_hbm, table_hbm, out_hbm, idx0_v, idx1_v, rows_v,
                      tab_sh, sem_in, sem_g, sem_out):
        sid = lax.axis_index("s")
        wid = sid * NUM_CORES + lax.axis_index("c")
        base = wid * B_PER_W
        cp_lab0 = pltpu.async_copy(
            labels_hbm.at[pl.ds(base, HALF)], idx0_v, sem_in)
        cp_lab1 = pltpu.async_copy(
            labels_hbm.at[pl.ds(base + HALF, HALF)], idx1_v, sem_in)

        @pl.when(sid == 0)
        def _():
            pltpu.sync_copy(table_hbm, tab_sh)

        cp_lab0.wait()
        cp_lab1.wait()
        plsc.subcore_barrier()
        g0 = pltpu.async_copy(tab_sh.at[idx0_v], rows_v.at[0], sem_g)
        g1 = pltpu.async_copy(tab_sh.at[idx1_v], rows_v.at[1], sem_g)
        g0.wait()
        o0 = pltpu.async_copy(
            rows_v.at[0], out_hbm.at[pl.ds(base, HALF)], sem_out)
        g1.wait()
        o1 = pltpu.async_copy(
            rows_v.at[1], out_hbm.at[pl.ds(base + HALF, HALF)], sem_out)
        o0.wait()
        o1.wait()

    return lookup_kernel


_lookup = _build()


def kernel(labels, table):
    out = _lookup(labels, table)
    return out.reshape(BATCH, 1, EMB_D)


# speedup vs baseline: 1.4707x; 1.0137x over previous
import functools

import jax
import jax.numpy as jnp
from jax import lax
from jax.experimental import pallas as pl
from jax.experimental.pallas import tpu as pltpu
from jax.experimental.pallas import tpu_sc as plsc

EMB_D = 128
BATCH = 16384
NUM_CORES = 2
NUM_SUBCORES = 16
NUM_WORKERS = NUM_CORES * NUM_SUBCORES
B_PER_W = BATCH // NUM_WORKERS
NSTAGE = 4
CHUNK = B_PER_W // NSTAGE


def _build():
    mesh = plsc.VectorSubcoreMesh(core_axis_name="c", subcore_axis_name="s")

    @functools.partial(
        pl.kernel,
        mesh=mesh,
        out_type=jax.ShapeDtypeStruct((BATCH, EMB_D), jnp.float32),
        scratch_types=(
            [pltpu.VMEM((CHUNK,), jnp.int32) for _ in range(NSTAGE)]
            + [
                pltpu.VMEM((NSTAGE, CHUNK, EMB_D), jnp.float32),
                pltpu.VMEM_SHARED((3, EMB_D), jnp.float32),
                pltpu.SemaphoreType.DMA,
                pltpu.SemaphoreType.DMA,
                pltpu.SemaphoreType.DMA,
            ]
        ),
    )
    def lookup_kernel(labels_hbm, table_hbm, out_hbm, *rest):
        idx = rest[:NSTAGE]
        rows_v, tab_sh, sem_in, sem_g, sem_out = rest[NSTAGE:]
        sid = lax.axis_index("s")
        wid = sid * NUM_CORES + lax.axis_index("c")
        base = wid * B_PER_W
        cp_labs = [
            pltpu.async_copy(
                labels_hbm.at[pl.ds(base + k * CHUNK, CHUNK)], idx[k], sem_in)
            for k in range(NSTAGE)
        ]

        @pl.when(sid == 0)
        def _():
            pltpu.sync_copy(table_hbm, tab_sh)

        for cp in cp_labs:
            cp.wait()
        plsc.subcore_barrier()
        gs = [
            pltpu.async_copy(tab_sh.at[idx[k]], rows_v.at[k], sem_g)
            for k in range(NSTAGE)
        ]
        outs = []
        for k in range(NSTAGE):
            gs[k].wait()
            outs.append(pltpu.async_copy(
                rows_v.at[k],
                out_hbm.at[pl.ds(base + k * CHUNK, CHUNK)], sem_out))
        for cp in outs:
            cp.wait()

    return lookup_kernel


_lookup = _build()


def kernel(labels, table):
    out = _lookup(labels, table)
    return out.reshape(BATCH, 1, EMB_D)
